# baseline (speedup 1.0000x reference)
import functools

import jax
import jax.numpy as jnp
from jax import lax
from jax.experimental import pallas as pl
from jax.experimental.pallas import tpu as pltpu
from jax.experimental.pallas import tpu_sc as plsc

VOCAB = 1000000
EMB_DIM = 64
OUT_DIM = 1000
BATCH = 4096

NC = 2
NS = 16
NW = NC * NS
ROWS_PER_W = BATCH // NW
L = 200
HALF = 100
NBUF = 3
VREGS = EMB_DIM // 16

VCHUNK = 128
NFULL = VOCAB // VCHUNK
VTAIL = VOCAB - NFULL * VCHUNK


def _sc_relayout(tblT, tail_packed):
    mesh = plsc.VectorSubcoreMesh(core_axis_name="c", subcore_axis_name="s")

    @functools.partial(
        pl.kernel,
        out_type=jax.ShapeDtypeStruct((VOCAB // 2, 128), jnp.float32),
        mesh=mesh,
        scratch_types=[
            pltpu.VMEM((2, EMB_DIM, VCHUNK), jnp.float32),
            pltpu.VMEM((2, VCHUNK // 2, 128), jnp.float32),
        ] + [pltpu.SemaphoreType.DMA] * 4,
        compiler_params=pltpu.CompilerParams(use_tc_tiling_on_sc=True,
                                             needs_layout_passes=False),
    )
    def k(t_hbm, tail_hbm, out_hbm, a_v, b_v, si0, si1, so0, so1):
        wid = lax.axis_index("s") * NC + lax.axis_index("c")
        sin = (si0, si1)
        sout = (so0, so1)
        iota = lax.iota(jnp.int32, 16)
        rowidx = [(16 * j + iota) >> 1 for j in range(8)]
        colbase = [((16 * j + iota) & 1) << 6 for j in range(8)]
        nk = pl.cdiv(NFULL - wid, NW)

        def in_copy(k_, s):
            c = wid + NW * k_
            return pltpu.make_async_copy(
                t_hbm.at[:, pl.ds(c * VCHUNK, VCHUNK)], a_v.at[s], sin[s])

        def out_copy(k_, s):
            c = wid + NW * k_
            return pltpu.make_async_copy(
                b_v.at[s], out_hbm.at[pl.ds(c * (VCHUNK // 2), VCHUNK // 2)],
                sout[s])

        for s in range(2):
            @pl.when(s < nk)
            def _():
                in_copy(s, s).start()

        def step(g, _):
            for s in range(2):
                k_ = g * 2 + s

                @pl.when(k_ < nk)
                def _():
                    in_copy(k_, s).wait()

                    @pl.when(k_ >= 2)
                    def _():
                        out_copy(k_ - 2, s).wait()

                    def trans(f, _):
                        for j in range(8):
                            x = a_v[s, f, pl.ds(16 * j, 16)]
                            plsc.store_scatter(
                                b_v.at[s], [rowidx[j], colbase[j] + f], x)
                        return ()

                    lax.fori_loop(0, EMB_DIM, trans, (), unroll=2)
                    out_copy(k_, s).start()

                    @pl.when(k_ + 2 < nk)
                    def _():
                        in_copy(k_ + 2, s).start()
            return ()

        lax.fori_loop(0, pl.cdiv(nk, 2), step, ())
        for s in range(2):
            @pl.when(s < nk)
            def _():
                pltpu.make_async_copy(
                    b_v.at[s], out_hbm.at[pl.ds(0, VCHUNK // 2)],
                    sout[s]).wait()

        @pl.when(wid == 0)
        def _():
            pltpu.sync_copy(tail_hbm, b_v.at[0, pl.ds(0, VTAIL // 2)])
            pltpu.sync_copy(b_v.at[0, pl.ds(0, VTAIL // 2)],
                            out_hbm.at[pl.ds(NFULL * VCHUNK // 2, VTAIL // 2)])

    return k(tblT, tail_packed)


def _sc_gather_sum(idx2d, table_lin):
    mesh = plsc.VectorSubcoreMesh(core_axis_name="c", subcore_axis_name="s")

    @functools.partial(
        pl.kernel,
        out_type=jax.ShapeDtypeStruct((BATCH, EMB_DIM), jnp.float32),
        mesh=mesh,
        scratch_types=[
            pltpu.VMEM((2 * ROWS_PER_W, HALF), jnp.int32),
            pltpu.VMEM((NBUF, L, EMB_DIM), jnp.float32),
            pltpu.VMEM((ROWS_PER_W, EMB_DIM), jnp.float32),
        ] + [pltpu.SemaphoreType.DMA] * NBUF,
        compiler_params=pltpu.CompilerParams(use_tc_tiling_on_sc=False),
    )
    def k(idx_hbm, table_hbm, out_hbm, idx_v, buf_v, acc_v, *sems):
        wid = lax.axis_index("s") * NC + lax.axis_index("c")
        pltpu.sync_copy(idx_hbm.at[pl.ds(wid * 2 * ROWS_PER_W, 2 * ROWS_PER_W)],
                        idx_v)

        def gather_copies(t, b):
            return [
                pltpu.make_async_copy(
                    table_hbm.at[idx_v.at[2 * t + h]],
                    buf_v.at[b, pl.ds(h * HALF, HALF)],
                    sems[b],
                )
                for h in range(2)
            ]

        for b in range(NBUF):
            for cp in gather_copies(b, b):
                cp.start()

        def sum_task(t, b):
            def body(i, vs):
                return tuple(
                    vs[c] + buf_v[b, i, pl.ds(16 * c, 16)]
                    for c in range(VREGS)
                )
            vs = lax.fori_loop(
                0, L, body,
                tuple(jnp.zeros((16,), jnp.float32) for _ in range(VREGS)),
                unroll=8)
            for c in range(VREGS):
                acc_v[t, pl.ds(16 * c, 16)] = vs[c]

        def outer(g, _):
            for b in range(NBUF):
                t = g * NBUF + b
                for cp in gather_copies(t, b):
                    cp.wait()
                sum_task(t, b)

                @pl.when(t + NBUF < ROWS_PER_W)
                def _():
                    for cp in gather_copies(t + NBUF, b):
                        cp.start()
            return ()

        lax.fori_loop(0, ROWS_PER_W // NBUF, outer, ())
        for t in range((ROWS_PER_W // NBUF) * NBUF, ROWS_PER_W):
            b = t % NBUF
            for cp in gather_copies(t, b):
                cp.wait()
            sum_task(t, b)

        pltpu.sync_copy(acc_v, out_hbm.at[pl.ds(wid * ROWS_PER_W, ROWS_PER_W)])

    return k(idx2d, table_lin)


def _tc_norm_linear_body(x_ref, w_ref, b_ref, o_ref):
    x = x_ref[...]
    ss = jnp.sum(x * x, axis=1, keepdims=True)
    inv = lax.rsqrt(jnp.maximum(ss, 1e-24))
    xn = x * inv
    o_ref[...] = lax.dot_general(
        w_ref[...], xn, (((1,), (1,)), ((), ())),
        preferred_element_type=jnp.float32) + b_ref[...]


def _tc_norm_linear_t(sums, W, b):
    BM = 1024
    return pl.pallas_call(
        _tc_norm_linear_body,
        grid=(BATCH // BM,),
        in_specs=[
            pl.BlockSpec((BM, EMB_DIM), lambda i: (i, 0)),
            pl.BlockSpec((OUT_DIM, EMB_DIM), lambda i: (0, 0)),
            pl.BlockSpec((OUT_DIM, 1), lambda i: (0, 0)),
        ],
        out_specs=pl.BlockSpec((OUT_DIM, BM), lambda i: (0, i)),
        out_shape=jax.ShapeDtypeStruct((OUT_DIM, BATCH), jnp.float32),
    )(sums, W, b.reshape(OUT_DIM, 1))


@jax.jit
def kernel(name_idxs, name_len, desc_idxs, desc_len, union_idxs, union_len,
           table, W, b):
    idx2d = union_idxs.astype(jnp.int32).reshape(2 * BATCH, HALF)
    tail_packed = table[NFULL * VCHUNK:, :].reshape(VTAIL // 2, 128)
    packed = _sc_relayout(table.T, tail_packed)
    table_lin = packed.reshape(VOCAB, EMB_DIM)
    sums = _sc_gather_sum(idx2d, table_lin)
    return _tc_norm_linear_t(sums, W, b).T
